# Initial kernel scaffold; baseline (speedup 1.0000x reference)
import jax
import jax.numpy as jnp
from jax.experimental import pallas as pl


def kernel(hidden_states, expert_indices, gate_up_w, down_w):
    raise NotImplementedError("write your pallas kernel here")


# speedup vs baseline: 1.0949x; 1.0949x over previous
import functools

import jax
import jax.numpy as jnp
from jax import lax
from jax.experimental import pallas as pl

NUM_EXPERTS = 8
D_MODEL = 1024
D_FF = 4096
T = 2048

FF_CHUNK = 512
ROW_SUB = 256
N_FF = D_FF // FF_CHUNK


def _dense_body(mask_ref, x_ref, wg_ref, wu_ref, wd_ref, out_ref):
    e = pl.program_id(0)
    ff = pl.program_id(1)

    @pl.when((e == 0) & (ff == 0))
    def _init():
        out_ref[...] = jnp.zeros_like(out_ref)

    lane_e = lax.broadcasted_iota(jnp.int32, (ROW_SUB, NUM_EXPERTS), 1) == e

    def sub(i, _):
        rows = pl.ds(i * ROW_SUB, ROW_SUB)
        m = jnp.sum(
            jnp.where(lane_e, mask_ref[rows, :], 0).astype(jnp.float32),
            axis=1, keepdims=True)
        xm = x_ref[rows, :] * m
        g = jnp.dot(xm, wg_ref[0], preferred_element_type=jnp.float32)
        u = jnp.dot(xm, wu_ref[0], preferred_element_type=jnp.float32)
        h = u * (g * jax.nn.sigmoid(g))
        out_ref[rows, :] += jnp.dot(h, wd_ref[0],
                                    preferred_element_type=jnp.float32)
        return 0

    lax.fori_loop(0, T // ROW_SUB, sub, 0)


def kernel(hidden_states, expert_indices, gate_up_w, down_w):
    grid = (NUM_EXPERTS, N_FF)
    return pl.pallas_call(
        _dense_body,
        grid=grid,
        in_specs=[
            pl.BlockSpec((T, NUM_EXPERTS), lambda e, f: (0, 0)),
            pl.BlockSpec((T, D_MODEL), lambda e, f: (0, 0)),
            pl.BlockSpec((1, D_MODEL, FF_CHUNK), lambda e, f: (e, 0, f)),
            pl.BlockSpec((1, D_MODEL, FF_CHUNK), lambda e, f: (e, 0, N_FF + f)),
            pl.BlockSpec((1, FF_CHUNK, D_MODEL), lambda e, f: (e, f, 0)),
        ],
        out_specs=pl.BlockSpec((T, D_MODEL), lambda e, f: (0, 0)),
        out_shape=jax.ShapeDtypeStruct((T, D_MODEL), jnp.float32),
    )(expert_indices, hidden_states, gate_up_w, gate_up_w, down_w)
